# Initial kernel scaffold; baseline (speedup 1.0000x reference)
import jax
import jax.numpy as jnp
from jax.experimental import pallas as pl


def kernel(features, edge_index, adj0_vals, adj1_vals, kernel):
    raise NotImplementedError("write your pallas kernel here")


# speedup vs baseline: 6.5100x; 6.5100x over previous
import functools

import jax
import jax.numpy as jnp
from jax import lax
from jax.experimental import pallas as pl
from jax.experimental.pallas import tpu as pltpu
from jax.experimental.pallas import tpu_sc as plsc

N = 10000
E = 320000
D_FEAT = 128
UNITS = 128
DIM = UNITS // 2
GAMMA = 1.0

NC = 2
NS = 16
LANES = 16
C = 80
EPT = E // NS
NCHUNK = EPT // C
ROWS_PT = N // NS

TC_BLOCK = 1000


def _tc_body(x_ref, w_ref, tab_ref, kl_ref):
    i = pl.program_id(0)
    h = jnp.dot(x_ref[...], w_ref[...], preferred_element_type=jnp.float32)
    m = h[:, :DIM]
    v = h[:, DIM:]
    mean = jnp.where(m > 0, m, jnp.exp(jnp.minimum(m, 0.0)) - 1.0)
    var = jnp.maximum(v, 0.0)
    att = jnp.exp(-GAMMA * var)
    tab_ref[0] = mean * att
    tab_ref[1] = var * (att * att)
    kl_part = 0.5 * jnp.sum(
        jnp.mean(jnp.square(mean) + var - jnp.log(1e-8 + var) - 1.0, axis=1)
    )

    @pl.when(i == 0)
    def _():
        kl_ref[0, 0] = 0.0

    kl_ref[0, 0] += kl_part


def _tc_stage(features, w):
    return pl.pallas_call(
        _tc_body,
        grid=(N // TC_BLOCK,),
        in_specs=[
            pl.BlockSpec((TC_BLOCK, D_FEAT), lambda i: (i, 0)),
            pl.BlockSpec((D_FEAT, UNITS), lambda i: (0, 0)),
        ],
        out_specs=[
            pl.BlockSpec((2, TC_BLOCK, DIM), lambda i: (0, i, 0)),
            pl.BlockSpec(
                block_shape=(1, 1),
                index_map=lambda i: (0, 0),
                memory_space=pltpu.SMEM,
            ),
        ],
        out_shape=[
            jax.ShapeDtypeStruct((2, N, DIM), jnp.float32),
            jax.ShapeDtypeStruct((1, 1), jnp.float32),
        ],
    )(features, w)


def _sc_kernel(tab_hbm, col_hbm, row_hbm, adj_hbm, z_hbm, out_hbm,
               acc_sh, col_v, row_v, adj_v, rows_v, sem0, sem1):
    cid = lax.axis_index("c")
    sid = lax.axis_index("s")

    pltpu.sync_copy(col_hbm.at[cid, sid], col_v)
    pltpu.sync_copy(row_hbm.at[sid], row_v)
    pltpu.sync_copy(adj_hbm.at[cid, sid], adj_v)
    pltpu.sync_copy(
        z_hbm.at[pl.ds(sid * ROWS_PT, ROWS_PT)],
        acc_sh.at[pl.ds(sid * ROWS_PT, ROWS_PT)],
    )
    plsc.subcore_barrier()

    sems = (sem0, sem1)

    def start_gather(j, b):
        pltpu.async_copy(tab_hbm.at[col_v.at[j]], rows_v.at[b], sems[b])

    def wait_gather(j, b):
        pltpu.make_async_copy(
            tab_hbm.at[col_v.at[j]], rows_v.at[b], sems[b]
        ).wait()

    def scale_chunk(j, b):
        buf = rows_v.at[b]

        def gbody(g, carry):
            base = g * LANES
            a_vec = adj_v[j, pl.ds(base, LANES)]
            for l in range(LANES):
                a = jnp.full((LANES,), a_vec[l], jnp.float32)
                e = base + l
                for f in range(DIM // LANES):
                    sl = pl.ds(f * LANES, LANES)
                    buf[e, sl] = buf[e, sl] * a
            return carry

        lax.fori_loop(0, C // LANES, gbody, None)

    start_gather(0, 0)

    def body(i, carry):
        j0 = i * 2
        j1 = j0 + 1
        start_gather(j1, 1)
        wait_gather(j0, 0)
        scale_chunk(j0, 0)
        pltpu.sync_copy(rows_v.at[0], acc_sh.at[row_v.at[j0]], add=True)

        @pl.when(j1 + 1 < NCHUNK)
        def _():
            start_gather(j1 + 1, 0)

        wait_gather(j1, 1)
        scale_chunk(j1, 1)
        pltpu.sync_copy(rows_v.at[1], acc_sh.at[row_v.at[j1]], add=True)
        return carry

    lax.fori_loop(0, NCHUNK // 2, body, None)

    plsc.subcore_barrier()
    pltpu.sync_copy(
        acc_sh.at[pl.ds(sid * ROWS_PT, ROWS_PT)],
        out_hbm.at[cid, pl.ds(sid * ROWS_PT, ROWS_PT)],
    )


def _sc_stage(tab2, col_all, row3, adj_all, zeros):
    mesh = plsc.VectorSubcoreMesh(core_axis_name="c", subcore_axis_name="s")
    run = functools.partial(
        pl.kernel,
        out_type=jax.ShapeDtypeStruct((NC, N, DIM), jnp.float32),
        mesh=mesh,
        scratch_types=[
            pltpu.VMEM_SHARED((N, DIM), jnp.float32),
            pltpu.VMEM((NCHUNK, C), jnp.int32),
            pltpu.VMEM((NCHUNK, C), jnp.int32),
            pltpu.VMEM((NCHUNK, C), jnp.float32),
            pltpu.VMEM((2, C, DIM), jnp.float32),
            pltpu.SemaphoreType.DMA,
            pltpu.SemaphoreType.DMA,
        ],
        compiler_params=pltpu.CompilerParams(use_tc_tiling_on_sc=False),
    )(_sc_kernel)
    return run(tab2, col_all, row3, adj_all, zeros)


def kernel(features, edge_index, adj0_vals, adj1_vals, kernel):
    tab, kl = _tc_stage(features, kernel)
    row = edge_index[0]
    col = edge_index[1]
    col_all = jnp.stack([col, col + N]).reshape(NC, NS, NCHUNK, C)
    row3 = row.reshape(NS, NCHUNK, C)
    adj_all = jnp.stack([adj0_vals, adj1_vals]).reshape(NC, NS, NCHUNK, C)
    zeros = jnp.zeros((N, DIM), jnp.float32)
    out2 = _sc_stage(tab.reshape(2 * N, DIM), col_all, row3, adj_all, zeros)
    output = jnp.concatenate([out2[0], out2[1]], axis=1)
    return (output, kl[0, 0])


# speedup vs baseline: 14.4467x; 2.2191x over previous
import functools

import jax
import jax.numpy as jnp
from jax import lax
from jax.experimental import pallas as pl
from jax.experimental.pallas import tpu as pltpu
from jax.experimental.pallas import tpu_sc as plsc

N = 10000
E = 320000
D_FEAT = 128
UNITS = 128
DIM = UNITS // 2
GAMMA = 1.0

NC = 2
NS = 16
LANES = 16
C = 80
EPT = E // NS
NBUF = 5
SB = 50
SEDGES = SB * C
NSUPER = EPT // SEDGES
ROWS_PT = N // NS

TC_BLOCK = 1000


def _tc_body(x_ref, w_ref, tab_ref, kl_ref):
    i = pl.program_id(0)
    h = jnp.dot(x_ref[...], w_ref[...], preferred_element_type=jnp.float32)
    m = h[:, :DIM]
    v = h[:, DIM:]
    mean = jnp.where(m > 0, m, jnp.exp(jnp.minimum(m, 0.0)) - 1.0)
    var = jnp.maximum(v, 0.0)
    att = jnp.exp(-GAMMA * var)
    tab_ref[0] = mean * att
    tab_ref[1] = var * (att * att)
    kl_part = 0.5 * jnp.sum(
        jnp.mean(jnp.square(mean) + var - jnp.log(1e-8 + var) - 1.0, axis=1)
    )

    @pl.when(i == 0)
    def _():
        kl_ref[0, 0] = 0.0

    kl_ref[0, 0] += kl_part


def _tc_stage(features, w):
    return pl.pallas_call(
        _tc_body,
        grid=(N // TC_BLOCK,),
        in_specs=[
            pl.BlockSpec((TC_BLOCK, D_FEAT), lambda i: (i, 0)),
            pl.BlockSpec((D_FEAT, UNITS), lambda i: (0, 0)),
        ],
        out_specs=[
            pl.BlockSpec((2, TC_BLOCK, DIM), lambda i: (0, i, 0)),
            pl.BlockSpec(
                block_shape=(1, 1),
                index_map=lambda i: (0, 0),
                memory_space=pltpu.SMEM,
            ),
        ],
        out_shape=[
            jax.ShapeDtypeStruct((2, N, DIM), jnp.float32),
            jax.ShapeDtypeStruct((1, 1), jnp.float32),
        ],
    )(features, w)


def _sc_kernel(tab_hbm, col_hbm, row_hbm, adj0_hbm, adj1_hbm, out_hbm,
               acc_sh, col_v, row_v, adj_v, rows_v, gsems, ssems):
    cid = lax.axis_index("c")
    sid = lax.axis_index("s")

    zero = jnp.zeros((LANES,), jnp.float32)

    def zfill(r, carry):
        for f in range(DIM // LANES):
            rows_v[0, r, pl.ds(f * LANES, LANES)] = zero
        return carry

    lax.fori_loop(0, C, zfill, None, unroll=4)
    for k in range(ROWS_PT // C):
        pltpu.sync_copy(
            rows_v.at[0], acc_sh.at[pl.ds(sid * ROWS_PT + k * C, C)]
        )
    rem = ROWS_PT % C
    pltpu.sync_copy(
        rows_v.at[0].at[pl.ds(0, rem)],
        acc_sh.at[pl.ds(sid * ROWS_PT + (ROWS_PT // C) * C, rem)],
    )

    plsc.subcore_barrier()

    def start_gather(j, b):
        pltpu.async_copy(
            tab_hbm.at[col_v.at[pl.ds(j * C, C)]], rows_v.at[b], gsems.at[b]
        )

    def wait_gather(j, b):
        pltpu.make_async_copy(
            tab_hbm.at[col_v.at[pl.ds(j * C, C)]], rows_v.at[b], gsems.at[b]
        ).wait()

    def start_scatter(j, b):
        pltpu.async_copy(
            rows_v.at[b], acc_sh.at[row_v.at[pl.ds(j * C, C)]], ssems.at[b],
            add=True,
        )

    def wait_scatter(j, b):
        pltpu.make_async_copy(
            rows_v.at[b], acc_sh.at[row_v.at[pl.ds(j * C, C)]], ssems.at[b]
        ).wait()

    def scale_chunk(j, b):
        buf = rows_v.at[b]

        def ebody(e, carry):
            idx = jnp.full((LANES,), j * C + e, jnp.int32)
            a = plsc.load_gather(adj_v, [idx])
            for f in range(DIM // LANES):
                sl = pl.ds(f * LANES, LANES)
                buf[e, sl] = buf[e, sl] * a
            return carry

        lax.fori_loop(0, C, ebody, None, unroll=8)

    def super_body(s, carry):
        sbase = sid * EPT + s * SEDGES
        pltpu.sync_copy(col_hbm.at[pl.ds(sbase, SEDGES)], col_v)
        pltpu.sync_copy(row_hbm.at[pl.ds(sbase, SEDGES)], row_v)

        @pl.when(cid == 0)
        def _():
            pltpu.sync_copy(adj0_hbm.at[pl.ds(sbase, SEDGES)], adj_v)

        @pl.when(cid == 1)
        def _():
            pltpu.sync_copy(adj1_hbm.at[pl.ds(sbase, SEDGES)], adj_v)
            offs = jnp.full((LANES,), N, jnp.int32)

            def add_off(g, c2):
                sl = pl.ds(g * LANES, LANES)
                col_v[sl] = col_v[sl] + offs
                return c2

            lax.fori_loop(0, SEDGES // LANES, add_off, None, unroll=8)

        for b in range(NBUF - 1):
            start_gather(b, b)

        def body(i, c2):
            for b in range(NBUF):
                j = i * NBUF + b
                bprev = (b - 1) % NBUF
                wait_gather(j, b)
                scale_chunk(j, b)
                start_scatter(j, b)

                @pl.when(j >= 1)
                def _():
                    wait_scatter(j - 1, bprev)

                @pl.when(j + NBUF - 1 < SB)
                def _():
                    start_gather(j + NBUF - 1, bprev)
            return c2

        lax.fori_loop(0, SB // NBUF, body, None)
        wait_scatter(SB - 1, (SB - 1) % NBUF)
        return carry

    lax.fori_loop(0, NSUPER, super_body, None)

    plsc.subcore_barrier()
    pltpu.sync_copy(
        acc_sh.at[pl.ds(sid * ROWS_PT, ROWS_PT)],
        out_hbm.at[pl.ds(sid * ROWS_PT, ROWS_PT), pl.ds(cid * DIM, DIM)],
    )


def _sc_stage(tab2, col, row, adj0, adj1):
    mesh = plsc.VectorSubcoreMesh(core_axis_name="c", subcore_axis_name="s")
    run = functools.partial(
        pl.kernel,
        out_type=jax.ShapeDtypeStruct((N, UNITS), jnp.float32),
        mesh=mesh,
        scratch_types=[
            pltpu.VMEM_SHARED((N, DIM), jnp.float32),
            pltpu.VMEM((SEDGES,), jnp.int32),
            pltpu.VMEM((SEDGES,), jnp.int32),
            pltpu.VMEM((SEDGES,), jnp.float32),
            pltpu.VMEM((NBUF, C, DIM), jnp.float32),
            pltpu.SemaphoreType.DMA((NBUF,)),
            pltpu.SemaphoreType.DMA((NBUF,)),
        ],
        compiler_params=pltpu.CompilerParams(
            use_tc_tiling_on_sc=False, needs_layout_passes=False
        ),
    )(_sc_kernel)
    return run(tab2, col, row, adj0, adj1)


def kernel(features, edge_index, adj0_vals, adj1_vals, kernel):
    tab, kl = _tc_stage(features, kernel)
    output = _sc_stage(
        tab.reshape(2 * N, DIM),
        edge_index[1], edge_index[0], adj0_vals, adj1_vals,
    )
    return (output, kl[0, 0])
